# baseline (device time: 19560 ns/iter reference)
import jax
import jax.numpy as jnp
from jax import lax
from jax.experimental import pallas as pl
from jax.experimental.pallas import tpu as pltpu

N_DEV = 4
E_LOCAL = 4
N_TOK = 512
D_IN = 256
D_OUT = 512
N_EXP = 16
N_CHUNK = 4
CW = D_OUT // N_CHUNK


def kernel(x, router_W, route_idx, expert_W):
    def body(x_ref, rw_ref, idx_ref, ew_ref, out_ref,
             sendA_ref, recvA_ref, sendB_ref, recvB_ref,
             send_sems, recv_sems):
        my = lax.axis_index("i")
        p_y = jnp.bitwise_xor(my, 1)
        p_x = 3 - my

        barrier_sem = pltpu.get_barrier_semaphore()
        for nbr in [p_y, p_x]:
            pl.semaphore_signal(
                barrier_sem, inc=1,
                device_id=(nbr,), device_id_type=pl.DeviceIdType.MESH,
            )
        pl.semaphore_wait(barrier_sem, 2)

        xf = x_ref[:, :]
        scores = jnp.dot(xf, rw_ref[:, :],
                         preferred_element_type=jnp.float32)
        s_max = jnp.max(scores, axis=-1, keepdims=True)
        p = jnp.exp(scores - s_max)
        p = p / jnp.sum(p, axis=-1, keepdims=True)

        idx0 = idx_ref[:, 0:1]
        idx1 = idx_ref[:, 1:2]
        eiota = lax.broadcasted_iota(jnp.int32, (N_TOK, N_EXP), 1)
        g0 = jnp.sum(jnp.where(eiota == idx0, p, 0.0), axis=1, keepdims=True)
        g1 = jnp.sum(jnp.where(eiota == idx1, p, 0.0), axis=1, keepdims=True)
        gs = g0 + g1
        w0 = g0 / gs
        w1 = g1 / gs

        xs_all = jnp.concatenate(
            [
                (xf * (jnp.where(idx0 == my * E_LOCAL + le, w0, 0.0)
                       + jnp.where(idx1 == my * E_LOCAL + le, w1, 0.0))
                 ).astype(jnp.bfloat16)
                for le in range(E_LOCAL)
            ],
            axis=1,
        )
        ew_all = ew_ref[...].reshape(E_LOCAL * D_IN, D_OUT).astype(jnp.bfloat16)

        partners_A = [p_y, p_x]
        partners_B = [p_x, p_y]

        partials = []
        rdmaA = []
        for c in range(N_CHUNK):
            pc = jnp.dot(xs_all, ew_all[:, c * CW:(c + 1) * CW],
                         preferred_element_type=jnp.float32)
            partials.append(pc)
            sendA_ref[c] = pc.astype(jnp.bfloat16)
            r = pltpu.make_async_remote_copy(
                src_ref=sendA_ref.at[c],
                dst_ref=recvA_ref.at[c],
                send_sem=send_sems.at[c],
                recv_sem=recv_sems.at[c],
                device_id=(partners_A[c % 2],),
                device_id_type=pl.DeviceIdType.MESH,
            )
            r.start()
            rdmaA.append(r)

        accs = []
        rdmaB = []
        for c in range(N_CHUNK):
            rdmaA[c].wait()
            a = partials[c] + recvA_ref[c].astype(jnp.float32)
            accs.append(a)
            sendB_ref[c] = a.astype(jnp.bfloat16)
            r = pltpu.make_async_remote_copy(
                src_ref=sendB_ref.at[c],
                dst_ref=recvB_ref.at[c],
                send_sem=send_sems.at[N_CHUNK + c],
                recv_sem=recv_sems.at[N_CHUNK + c],
                device_id=(partners_B[c % 2],),
                device_id_type=pl.DeviceIdType.MESH,
            )
            r.start()
            rdmaB.append(r)

        for c in range(N_CHUNK):
            rdmaB[c].wait()
            out_ref[:, c * CW:(c + 1) * CW] = (
                accs[c] + recvB_ref[c].astype(jnp.float32))

    return pl.pallas_call(
        body,
        out_shape=jax.ShapeDtypeStruct((N_TOK, D_OUT), jnp.float32),
        in_specs=[
            pl.BlockSpec(memory_space=pltpu.VMEM),
            pl.BlockSpec(memory_space=pltpu.VMEM),
            pl.BlockSpec(memory_space=pltpu.VMEM),
            pl.BlockSpec(memory_space=pltpu.VMEM),
        ],
        out_specs=pl.BlockSpec(memory_space=pltpu.VMEM),
        scratch_shapes=[
            pltpu.VMEM((N_CHUNK, N_TOK, CW), jnp.bfloat16),
            pltpu.VMEM((N_CHUNK, N_TOK, CW), jnp.bfloat16),
            pltpu.VMEM((N_CHUNK, N_TOK, CW), jnp.bfloat16),
            pltpu.VMEM((N_CHUNK, N_TOK, CW), jnp.bfloat16),
            pltpu.SemaphoreType.DMA((2 * N_CHUNK,)),
            pltpu.SemaphoreType.DMA((2 * N_CHUNK,)),
        ],
        compiler_params=pltpu.CompilerParams(collective_id=0),
    )(x, router_W, route_idx, expert_W)


# device time: 19322 ns/iter; 1.0123x vs baseline; 1.0123x over previous
import jax
import jax.numpy as jnp
from jax import lax
from jax.experimental import pallas as pl
from jax.experimental.pallas import tpu as pltpu

N_DEV = 4
E_LOCAL = 4
N_TOK = 512
D_IN = 256
D_OUT = 512
N_EXP = 16
N_HALF = 2
HW = D_OUT // N_HALF


def kernel(x, router_W, route_idx, expert_W):
    def body(x_ref, rw_ref, idx_ref, ew_ref, out_ref,
             sendA_ref, recvA_ref, sendB_ref, recvB_ref,
             send_sems, recv_sems):
        my = lax.axis_index("i")
        p_y = jnp.bitwise_xor(my, 1)
        p_x = 3 - my

        barrier_sem = pltpu.get_barrier_semaphore()
        for nbr in [p_y, p_x]:
            pl.semaphore_signal(
                barrier_sem, inc=1,
                device_id=(nbr,), device_id_type=pl.DeviceIdType.MESH,
            )

        xf = x_ref[:, :]
        scores = jnp.dot(xf, rw_ref[:, :],
                         preferred_element_type=jnp.float32)
        s_max = jnp.max(scores, axis=-1, keepdims=True)
        p = jnp.exp(scores - s_max)
        p = p / jnp.sum(p, axis=-1, keepdims=True)

        idx0 = idx_ref[:, 0:1]
        idx1 = idx_ref[:, 1:2]
        eiota = lax.broadcasted_iota(jnp.int32, (N_TOK, N_EXP), 1)
        g0 = jnp.sum(jnp.where(eiota == idx0, p, 0.0), axis=1, keepdims=True)
        g1 = jnp.sum(jnp.where(eiota == idx1, p, 0.0), axis=1, keepdims=True)
        gs = g0 + g1
        w0 = g0 / gs
        w1 = g1 / gs

        xs_all = jnp.concatenate(
            [
                (xf * (jnp.where(idx0 == my * E_LOCAL + le, w0, 0.0)
                       + jnp.where(idx1 == my * E_LOCAL + le, w1, 0.0))
                 ).astype(jnp.bfloat16)
                for le in range(E_LOCAL)
            ],
            axis=1,
        )
        ew_all = ew_ref[...].reshape(E_LOCAL * D_IN, D_OUT).astype(jnp.bfloat16)

        partners_A = [p_y, p_x]
        partners_B = [p_x, p_y]

        partials = []
        rdmaA = []
        for h in range(N_HALF):
            pc = jnp.dot(xs_all, ew_all[:, h * HW:(h + 1) * HW],
                         preferred_element_type=jnp.float32)
            partials.append(pc)
            sendA_ref[h] = pc.astype(jnp.bfloat16)
            if h == 0:
                pl.semaphore_wait(barrier_sem, 2)
            r = pltpu.make_async_remote_copy(
                src_ref=sendA_ref.at[h],
                dst_ref=recvA_ref.at[h],
                send_sem=send_sems.at[h],
                recv_sem=recv_sems.at[h],
                device_id=(partners_A[h],),
                device_id_type=pl.DeviceIdType.MESH,
            )
            r.start()
            rdmaA.append(r)

        accs = []
        rdmaB = []
        for h in range(N_HALF):
            rdmaA[h].wait()
            a = partials[h] + recvA_ref[h].astype(jnp.float32)
            accs.append(a)
            sendB_ref[h] = a.astype(jnp.bfloat16)
            r = pltpu.make_async_remote_copy(
                src_ref=sendB_ref.at[h],
                dst_ref=recvB_ref.at[h],
                send_sem=send_sems.at[N_HALF + h],
                recv_sem=recv_sems.at[N_HALF + h],
                device_id=(partners_B[h],),
                device_id_type=pl.DeviceIdType.MESH,
            )
            r.start()
            rdmaB.append(r)

        for h in range(N_HALF):
            rdmaB[h].wait()
            out_ref[:, h * HW:(h + 1) * HW] = (
                accs[h] + recvB_ref[h].astype(jnp.float32))

    return pl.pallas_call(
        body,
        out_shape=jax.ShapeDtypeStruct((N_TOK, D_OUT), jnp.float32),
        in_specs=[
            pl.BlockSpec(memory_space=pltpu.VMEM),
            pl.BlockSpec(memory_space=pltpu.VMEM),
            pl.BlockSpec(memory_space=pltpu.VMEM),
            pl.BlockSpec(memory_space=pltpu.VMEM),
        ],
        out_specs=pl.BlockSpec(memory_space=pltpu.VMEM),
        scratch_shapes=[
            pltpu.VMEM((N_HALF, N_TOK, HW), jnp.bfloat16),
            pltpu.VMEM((N_HALF, N_TOK, HW), jnp.bfloat16),
            pltpu.VMEM((N_HALF, N_TOK, HW), jnp.bfloat16),
            pltpu.VMEM((N_HALF, N_TOK, HW), jnp.bfloat16),
            pltpu.SemaphoreType.DMA((2 * N_HALF,)),
            pltpu.SemaphoreType.DMA((2 * N_HALF,)),
        ],
        compiler_params=pltpu.CompilerParams(collective_id=0),
    )(x, router_W, route_idx, expert_W)


# device time: 19160 ns/iter; 1.0209x vs baseline; 1.0085x over previous
import jax
import jax.numpy as jnp
from jax import lax
from jax.experimental import pallas as pl
from jax.experimental.pallas import tpu as pltpu

N_DEV = 4
E_LOCAL = 4
N_TOK = 512
D_IN = 256
D_OUT = 512
N_EXP = 16
N_HALF = 2
HW = D_OUT // N_HALF


def kernel(x, router_W, route_idx, expert_W):
    def body(x_ref, rw_ref, idx_ref, ew_ref, out_ref,
             sendA_ref, recvA_ref, sendB_ref, recvB_ref,
             send_sems, recv_sems):
        my = lax.axis_index("i")
        p_y = jnp.bitwise_xor(my, 1)
        p_x = 3 - my

        barrier_sem = pltpu.get_barrier_semaphore()
        for nbr in [p_y, p_x]:
            pl.semaphore_signal(
                barrier_sem, inc=1,
                device_id=(nbr,), device_id_type=pl.DeviceIdType.MESH,
            )
        pl.semaphore_wait(barrier_sem, 2)

        xf = x_ref[:, :]
        scores = jnp.dot(xf, rw_ref[:, :],
                         preferred_element_type=jnp.float32)
        s_max = jnp.max(scores, axis=-1, keepdims=True)
        p = jnp.exp(scores - s_max)
        p = p / jnp.sum(p, axis=-1, keepdims=True)

        idx0 = idx_ref[:, 0:1]
        idx1 = idx_ref[:, 1:2]
        eiota = lax.broadcasted_iota(jnp.int32, (N_TOK, N_EXP), 1)
        g0 = jnp.sum(jnp.where(eiota == idx0, p, 0.0), axis=1, keepdims=True)
        g1 = jnp.sum(jnp.where(eiota == idx1, p, 0.0), axis=1, keepdims=True)
        gs = g0 + g1
        w0 = g0 / gs
        w1 = g1 / gs

        xs_all = jnp.concatenate(
            [
                (xf * (jnp.where(idx0 == my * E_LOCAL + le, w0, 0.0)
                       + jnp.where(idx1 == my * E_LOCAL + le, w1, 0.0))
                 ).astype(jnp.bfloat16)
                for le in range(E_LOCAL)
            ],
            axis=1,
        )
        ew_all = ew_ref[...].reshape(E_LOCAL * D_IN, D_OUT).astype(jnp.bfloat16)

        partners_A = [p_y, p_x]
        partners_B = [p_x, p_y]

        partials = []
        rdmaA = []
        for h in range(N_HALF):
            pc = jnp.dot(xs_all, ew_all[:, h * HW:(h + 1) * HW],
                         preferred_element_type=jnp.float32)
            partials.append(pc)
            sendA_ref[h] = pc.astype(jnp.bfloat16)
            r = pltpu.make_async_remote_copy(
                src_ref=sendA_ref.at[h],
                dst_ref=recvA_ref.at[h],
                send_sem=send_sems.at[h],
                recv_sem=recv_sems.at[h],
                device_id=(partners_A[h],),
                device_id_type=pl.DeviceIdType.MESH,
            )
            r.start()
            rdmaA.append(r)

        accs = []
        rdmaB = []
        for h in range(N_HALF):
            rdmaA[h].wait()
            a = partials[h] + recvA_ref[h].astype(jnp.float32)
            accs.append(a)
            sendB_ref[h] = a.astype(jnp.bfloat16)
            r = pltpu.make_async_remote_copy(
                src_ref=sendB_ref.at[h],
                dst_ref=recvB_ref.at[h],
                send_sem=send_sems.at[N_HALF + h],
                recv_sem=recv_sems.at[N_HALF + h],
                device_id=(partners_B[h],),
                device_id_type=pl.DeviceIdType.MESH,
            )
            r.start()
            rdmaB.append(r)

        for h in range(N_HALF):
            rdmaB[h].wait()
            out_ref[:, h * HW:(h + 1) * HW] = (
                accs[h] + recvB_ref[h].astype(jnp.float32))

    return pl.pallas_call(
        body,
        out_shape=jax.ShapeDtypeStruct((N_TOK, D_OUT), jnp.float32),
        in_specs=[
            pl.BlockSpec(memory_space=pltpu.VMEM),
            pl.BlockSpec(memory_space=pltpu.VMEM),
            pl.BlockSpec(memory_space=pltpu.VMEM),
            pl.BlockSpec(memory_space=pltpu.VMEM),
        ],
        out_specs=pl.BlockSpec(memory_space=pltpu.VMEM),
        scratch_shapes=[
            pltpu.VMEM((N_HALF, N_TOK, HW), jnp.bfloat16),
            pltpu.VMEM((N_HALF, N_TOK, HW), jnp.bfloat16),
            pltpu.VMEM((N_HALF, N_TOK, HW), jnp.bfloat16),
            pltpu.VMEM((N_HALF, N_TOK, HW), jnp.bfloat16),
            pltpu.SemaphoreType.DMA((2 * N_HALF,)),
            pltpu.SemaphoreType.DMA((2 * N_HALF,)),
        ],
        compiler_params=pltpu.CompilerParams(collective_id=0),
    )(x, router_W, route_idx, expert_W)
